# Initial kernel scaffold; baseline (speedup 1.0000x reference)
import jax
import jax.numpy as jnp
from jax.experimental import pallas as pl


def kernel(input):
    raise NotImplementedError("write your pallas kernel here")


# speedup vs baseline: 6.7050x; 6.7050x over previous
import jax
import jax.numpy as jnp
from jax.experimental import pallas as pl

_NUM_ANCHORS = 3
_NUM_CLASSES = 80
_ATTRS = 5 + _NUM_CLASSES
_H = 52
_W = 52
_S = _H * _W
_STRIDE = 8.0
_ANCHOR_W = (10.0, 16.0, 33.0)
_ANCHOR_H = (13.0, 30.0, 23.0)


def _decode_kernel(in_ref, out_ref):
    col = jax.lax.broadcasted_iota(jnp.int32, (1, _S), 1)
    gx = (col % _W).astype(jnp.float32)
    gy = (col // _W).astype(jnp.float32)

    for a in range(_NUM_ANCHORS):
        blk = in_ref[0, a * _ATTRS:(a + 1) * _ATTRS, :]
        sig = jax.nn.sigmoid(blk)
        bx = (sig[0:1] + gx) * _STRIDE
        by = (sig[1:2] + gy) * _STRIDE
        bw = jnp.exp(blk[2:3]) * _ANCHOR_W[a]
        bh = jnp.exp(blk[3:4]) * _ANCHOR_H[a]
        dec = jnp.concatenate([bx, by, bw, bh, sig[4:]], axis=0)
        out_ref[0, a * _S:(a + 1) * _S, :] = dec.T


def kernel(input):
    bs = input.shape[0]
    flat = input.reshape(bs, _NUM_ANCHORS * _ATTRS, _S)
    out = pl.pallas_call(
        _decode_kernel,
        grid=(bs,),
        in_specs=[pl.BlockSpec((1, _NUM_ANCHORS * _ATTRS, _S), lambda b: (b, 0, 0))],
        out_specs=pl.BlockSpec((1, _NUM_ANCHORS * _S, _ATTRS), lambda b: (b, 0, 0)),
        out_shape=jax.ShapeDtypeStruct((bs, _NUM_ANCHORS * _S, _ATTRS), jnp.float32),
    )(flat)
    return out
